# baseline (speedup 1.0000x reference)
import functools

import jax
import jax.numpy as jnp
from jax import lax
from jax.experimental import pallas as pl
from jax.experimental.pallas import tpu as pltpu
from jax.experimental.pallas import tpu_sc as plsc

_NC = 2
_NS = 16
_NW = _NC * _NS
_GR = 40


def _gather_patches(seq1M, seq2M, pats_aligned):
    B, L, D = seq1M.shape
    NPAIR = pats_aligned.shape[0] // 2
    P = NPAIR // B

    slabs = (pats_aligned // 8).astype(jnp.uint32).reshape(-1, 4)
    packed = (slabs[:, 0] | (slabs[:, 1] << 8) | (slabs[:, 2] << 16)
              | (slabs[:, 3] << 24)).astype(jnp.int32)
    pvals = [packed[j] for j in range(packed.shape[0])]

    def unpack(j):
        word = pvals[j // 4]
        byte = (word >> (8 * (j % 4))) & 0xFF
        return pl.multiple_of(byte * 8, 8)

    def body(t1, t2, o1, o2, sem):
        handles = []
        for k in range(NPAIR):
            b = k // P
            p0 = unpack(2 * k + 0)
            p1 = unpack(2 * k + 1)
            handles.append(
                pltpu.async_copy(t1.at[b, pl.ds(p0, _GR)], o1.at[k], sem)
            )
            handles.append(
                pltpu.async_copy(t2.at[b, pl.ds(p1, _GR)], o2.at[k], sem)
            )
        for h in handles:
            h.wait()

    mesh = plsc.ScalarSubcoreMesh(axis_name="c", num_cores=1)
    out_t = [
        jax.ShapeDtypeStruct((NPAIR, _GR, D), jnp.float32),
        jax.ShapeDtypeStruct((NPAIR, _GR, D), jnp.float32),
    ]
    o1, o2 = pl.kernel(
        body,
        mesh=mesh,
        out_type=out_t,
        scratch_types=[pltpu.SemaphoreType.DMA],
    )(seq1M, seq2M)
    return o1.reshape(B, P, _GR, D), o2.reshape(B, P, _GR, D)


def _asm_body(P, PS, D, offs_ref, r1_ref, r2_ref, geo_ref, out_ref):
    b = pl.program_id(0)
    for i in range(P):
        o0 = offs_ref[(b * P + i) * 2 + 0]
        o1 = offs_ref[(b * P + i) * 2 + 1]
        r1 = r1_ref[0, i, pl.ds(o0, PS), :]
        r2 = r2_ref[0, i, pl.ds(o1, PS), :]
        g = geo_ref[0, i]
        blk = jnp.concatenate(
            [
                jnp.broadcast_to(r1[None, :, :], (PS, PS, D)),
                jnp.broadcast_to(r2[:, None, :], (PS, PS, D)),
                g[:, :, None],
            ],
            axis=2,
        )
        out_ref[0, i] = blk


def _assemble(r1s, r2s, geo, offs_flat, *, interpret=False):
    B, P, GR, D = r1s.shape
    PS = geo.shape[2]
    F = 2 * D + 1
    grid_spec = pltpu.PrefetchScalarGridSpec(
        num_scalar_prefetch=1,
        grid=(B,),
        in_specs=[
            pl.BlockSpec((1, P, GR, D), lambda b, offs: (b, 0, 0, 0)),
            pl.BlockSpec((1, P, GR, D), lambda b, offs: (b, 0, 0, 0)),
            pl.BlockSpec((1, P, PS, PS), lambda b, offs: (b, 0, 0, 0)),
        ],
        out_specs=pl.BlockSpec(
            (1, P, PS, PS, F), lambda b, offs: (b, 0, 0, 0, 0)
        ),
    )
    return pl.pallas_call(
        functools.partial(_asm_body, P, PS, D),
        grid_spec=grid_spec,
        out_shape=jax.ShapeDtypeStruct((B, P, PS, PS, F), jnp.float32),
        interpret=interpret,
    )(offs_flat, r1s, r2s, geo)


def kernel(seq1M, seq2M, patches, geo):
    B, L, D = seq1M.shape
    _, P, _ = patches.shape
    pats = patches.reshape(B * P * 2).astype(jnp.int32)
    pats_aligned = jnp.minimum((pats // 8) * 8, L - _GR)
    offs_flat = pats - pats_aligned
    r1s, r2s = _gather_patches(seq1M, seq2M, pats_aligned)
    return _assemble(r1s, r2s, geo, offs_flat)


# speedup vs baseline: 2.0205x; 2.0205x over previous
import functools

import jax
import jax.numpy as jnp
from jax.experimental import pallas as pl
from jax.experimental.pallas import tpu as pltpu

_GR = 40


def _body(P, PS, D, pat_ref, seq1_hbm, seq2_hbm, geo_ref, out_ref,
          scr, sems):
    b = pl.program_id(0)
    copies = []
    for i in range(P):
        pa0 = pl.multiple_of(pat_ref[(b * P + i) * 4 + 0], 8)
        pa1 = pl.multiple_of(pat_ref[(b * P + i) * 4 + 1], 8)
        copies.append(pltpu.make_async_copy(
            seq1_hbm.at[b, pl.ds(pa0, _GR)], scr.at[0, i], sems.at[0, i]))
        copies.append(pltpu.make_async_copy(
            seq2_hbm.at[b, pl.ds(pa1, _GR)], scr.at[1, i], sems.at[1, i]))
    for c in copies:
        c.start()
    for c in copies:
        c.wait()
    for i in range(P):
        o0 = pat_ref[(b * P + i) * 4 + 2]
        o1 = pat_ref[(b * P + i) * 4 + 3]
        r1 = scr[0, i, pl.ds(o0, PS), :]
        r2 = scr[1, i, pl.ds(o1, PS), :]
        g = geo_ref[0, i]
        blk = jnp.concatenate(
            [
                jnp.broadcast_to(r1[None, :, :], (PS, PS, D)),
                jnp.broadcast_to(r2[:, None, :], (PS, PS, D)),
                g[:, :, None],
            ],
            axis=2,
        )
        out_ref[0, i] = blk


def _make_blocks(seq1M, seq2M, pat_meta, geo, *, interpret=False):
    B, L, D = seq1M.shape
    _, P, PS, _ = geo.shape
    F = 2 * D + 1

    grid_spec = pltpu.PrefetchScalarGridSpec(
        num_scalar_prefetch=1,
        grid=(B,),
        in_specs=[
            pl.BlockSpec(memory_space=pl.ANY),
            pl.BlockSpec(memory_space=pl.ANY),
            pl.BlockSpec((1, P, PS, PS), lambda b, pat: (b, 0, 0, 0)),
        ],
        out_specs=pl.BlockSpec(
            (1, P, PS, PS, F), lambda b, pat: (b, 0, 0, 0, 0)
        ),
        scratch_shapes=[
            pltpu.VMEM((2, P, _GR, D), jnp.float32),
            pltpu.SemaphoreType.DMA((2, P)),
        ],
    )
    return pl.pallas_call(
        functools.partial(_body, P, PS, D),
        grid_spec=grid_spec,
        out_shape=jax.ShapeDtypeStruct((B, P, PS, PS, F), jnp.float32),
        interpret=interpret,
    )(pat_meta, seq1M, seq2M, geo)


def kernel(seq1M, seq2M, patches, geo):
    B, L, D = seq1M.shape
    _, P, _ = patches.shape
    pats = patches.reshape(B * P, 2).astype(jnp.int32)
    pats_aligned = jnp.minimum((pats // 8) * 8, L - _GR)
    offs = pats - pats_aligned
    pat_meta = jnp.concatenate([pats_aligned, offs], axis=1).reshape(-1)
    return _make_blocks(seq1M, seq2M, pat_meta, geo)


# speedup vs baseline: 2.3486x; 1.1624x over previous
import functools

import jax
import jax.numpy as jnp
from jax import lax
from jax.experimental import pallas as pl
from jax.experimental.pallas import tpu as pltpu

_GR = 40


def _copies(P, pat_ref, seq1_hbm, seq2_hbm, scr, sems, b, slot):
    cs = []
    for i in range(P):
        pa0 = pl.multiple_of(pat_ref[(b * P + i) * 4 + 0], 8)
        pa1 = pl.multiple_of(pat_ref[(b * P + i) * 4 + 1], 8)
        cs.append(pltpu.make_async_copy(
            seq1_hbm.at[b, pl.ds(pa0, _GR)], scr.at[slot, 0, i],
            sems.at[slot, 0, i]))
        cs.append(pltpu.make_async_copy(
            seq2_hbm.at[b, pl.ds(pa1, _GR)], scr.at[slot, 1, i],
            sems.at[slot, 1, i]))
    return cs


def _body(P, PS, D, pat_ref, seq1_hbm, seq2_hbm, geo_ref, out_ref,
          scr, sems):
    b = pl.program_id(0)
    B = pl.num_programs(0)
    slot = lax.rem(b, 2)

    @pl.when(b == 0)
    def _prologue():
        for c in _copies(P, pat_ref, seq1_hbm, seq2_hbm, scr, sems, 0, 0):
            c.start()

    @pl.when(b + 1 < B)
    def _prefetch_next():
        for c in _copies(P, pat_ref, seq1_hbm, seq2_hbm, scr, sems,
                         b + 1, lax.rem(b + 1, 2)):
            c.start()

    for c in _copies(P, pat_ref, seq1_hbm, seq2_hbm, scr, sems, b, slot):
        c.wait()

    for i in range(P):
        o0 = pat_ref[(b * P + i) * 4 + 2]
        o1 = pat_ref[(b * P + i) * 4 + 3]
        r1 = scr[slot, 0, i, pl.ds(o0, PS), :]
        r2 = scr[slot, 1, i, pl.ds(o1, PS), :]
        g = geo_ref[0, i]
        blk = jnp.concatenate(
            [
                jnp.broadcast_to(r1[None, :, :], (PS, PS, D)),
                jnp.broadcast_to(r2[:, None, :], (PS, PS, D)),
                g[:, :, None],
            ],
            axis=2,
        )
        out_ref[0, i] = blk


def _make_blocks(seq1M, seq2M, pat_meta, geo, *, interpret=False):
    B, L, D = seq1M.shape
    _, P, PS, _ = geo.shape
    F = 2 * D + 1

    grid_spec = pltpu.PrefetchScalarGridSpec(
        num_scalar_prefetch=1,
        grid=(B,),
        in_specs=[
            pl.BlockSpec(memory_space=pl.ANY),
            pl.BlockSpec(memory_space=pl.ANY),
            pl.BlockSpec((1, P, PS, PS), lambda b, pat: (b, 0, 0, 0)),
        ],
        out_specs=pl.BlockSpec(
            (1, P, PS, PS, F), lambda b, pat: (b, 0, 0, 0, 0)
        ),
        scratch_shapes=[
            pltpu.VMEM((2, 2, P, _GR, D), jnp.float32),
            pltpu.SemaphoreType.DMA((2, 2, P)),
        ],
    )
    return pl.pallas_call(
        functools.partial(_body, P, PS, D),
        grid_spec=grid_spec,
        out_shape=jax.ShapeDtypeStruct((B, P, PS, PS, F), jnp.float32),
        interpret=interpret,
    )(pat_meta, seq1M, seq2M, geo)


def kernel(seq1M, seq2M, patches, geo):
    B, L, D = seq1M.shape
    _, P, _ = patches.shape
    pats = patches.reshape(B * P, 2).astype(jnp.int32)
    pats_aligned = jnp.minimum((pats // 8) * 8, L - _GR)
    offs = pats - pats_aligned
    pat_meta = jnp.concatenate([pats_aligned, offs], axis=1).reshape(-1)
    return _make_blocks(seq1M, seq2M, pat_meta, geo)


# speedup vs baseline: 2.4101x; 1.0262x over previous
import functools

import jax
import jax.numpy as jnp
from jax import lax
from jax.experimental import pallas as pl
from jax.experimental.pallas import tpu as pltpu

_GR = 40
_NB = 2


def _copies(NB, P, pat_ref, seq1_hbm, seq2_hbm, scr, sems, s, slot):
    cs = []
    for bb in range(NB):
        b = s * NB + bb
        for i in range(P):
            pa0 = pl.multiple_of(pat_ref[(b * P + i) * 4 + 0], 8)
            pa1 = pl.multiple_of(pat_ref[(b * P + i) * 4 + 1], 8)
            cs.append(pltpu.make_async_copy(
                seq1_hbm.at[b, pl.ds(pa0, _GR)], scr.at[slot, bb, 0, i],
                sems.at[slot, bb, 0, i]))
            cs.append(pltpu.make_async_copy(
                seq2_hbm.at[b, pl.ds(pa1, _GR)], scr.at[slot, bb, 1, i],
                sems.at[slot, bb, 1, i]))
    return cs


def _body(NB, P, PS, D, pat_ref, seq1_hbm, seq2_hbm, geo_ref, out_ref,
          scr, sems):
    s = pl.program_id(0)
    NS = pl.num_programs(0)
    slot = lax.rem(s, 2)

    @pl.when(s == 0)
    def _prologue():
        for c in _copies(NB, P, pat_ref, seq1_hbm, seq2_hbm, scr, sems, 0, 0):
            c.start()

    @pl.when(s + 1 < NS)
    def _prefetch_next():
        for c in _copies(NB, P, pat_ref, seq1_hbm, seq2_hbm, scr, sems,
                         s + 1, lax.rem(s + 1, 2)):
            c.start()

    for c in _copies(NB, P, pat_ref, seq1_hbm, seq2_hbm, scr, sems, s, slot):
        c.wait()

    for bb in range(NB):
        b = s * NB + bb
        for i in range(P):
            o0 = pat_ref[(b * P + i) * 4 + 2]
            o1 = pat_ref[(b * P + i) * 4 + 3]
            r1 = scr[slot, bb, 0, i, pl.ds(o0, PS), :]
            r2 = scr[slot, bb, 1, i, pl.ds(o1, PS), :]
            g = geo_ref[bb, i]
            blk = jnp.concatenate(
                [
                    jnp.broadcast_to(r1[None, :, :], (PS, PS, D)),
                    jnp.broadcast_to(r2[:, None, :], (PS, PS, D)),
                    g[:, :, None],
                ],
                axis=2,
            )
            out_ref[bb, i] = blk


def _make_blocks(seq1M, seq2M, pat_meta, geo, *, interpret=False):
    B, L, D = seq1M.shape
    _, P, PS, _ = geo.shape
    F = 2 * D + 1
    NB = _NB

    grid_spec = pltpu.PrefetchScalarGridSpec(
        num_scalar_prefetch=1,
        grid=(B // NB,),
        in_specs=[
            pl.BlockSpec(memory_space=pl.ANY),
            pl.BlockSpec(memory_space=pl.ANY),
            pl.BlockSpec((NB, P, PS, PS), lambda s, pat: (s, 0, 0, 0)),
        ],
        out_specs=pl.BlockSpec(
            (NB, P, PS, PS, F), lambda s, pat: (s, 0, 0, 0, 0)
        ),
        scratch_shapes=[
            pltpu.VMEM((2, NB, 2, P, _GR, D), jnp.float32),
            pltpu.SemaphoreType.DMA((2, NB, 2, P)),
        ],
    )
    return pl.pallas_call(
        functools.partial(_body, NB, P, PS, D),
        grid_spec=grid_spec,
        out_shape=jax.ShapeDtypeStruct((B, P, PS, PS, F), jnp.float32),
        interpret=interpret,
    )(pat_meta, seq1M, seq2M, geo)


def kernel(seq1M, seq2M, patches, geo):
    B, L, D = seq1M.shape
    _, P, _ = patches.shape
    pats = patches.reshape(B * P, 2).astype(jnp.int32)
    pats_aligned = jnp.minimum((pats // 8) * 8, L - _GR)
    offs = pats - pats_aligned
    pat_meta = jnp.concatenate([pats_aligned, offs], axis=1).reshape(-1)
    return _make_blocks(seq1M, seq2M, pat_meta, geo)
